# Initial kernel scaffold; baseline (speedup 1.0000x reference)
import jax
import jax.numpy as jnp
from jax import lax
from jax.experimental import pallas as pl
from jax.experimental.pallas import tpu as pltpu
from jax.experimental.pallas import tpu_sc as plsc

N = 10000
E = 320000
F = 128
C = 47

NC = 2
NS = 16
NW = NC * NS
EW = E // NW
CHUNK = 80
NCHUNK = EW // CHUNK
NPAD = 10240
RPT = NPAD // NS


def _make_agg(with_counts):
    mesh = plsc.VectorSubcoreMesh(
        core_axis_name="c", subcore_axis_name="s",
        num_cores=NC, num_subcores=NS)
    out_type = [jax.ShapeDtypeStruct((NC, NPAD, F), jnp.float32)]
    scratch = [
        pltpu.VMEM((NCHUNK, CHUNK), jnp.int32),
        pltpu.VMEM((NCHUNK, CHUNK), jnp.int32),
        pltpu.VMEM((CHUNK, F), jnp.float32),
        pltpu.VMEM_SHARED((NPAD, F), jnp.float32),
    ]
    if with_counts:
        out_type.append(jax.ShapeDtypeStruct((NC, NPAD, 16), jnp.float32))
        scratch += [
            pltpu.VMEM((CHUNK, 16), jnp.float32),
            pltpu.VMEM_SHARED((NPAD, 16), jnp.float32),
        ]

    def body(table, src3, dst3, zrow, zcnt, ones_h, *rest):
        if with_counts:
            out_sum, out_cnt, src_v, dst_v, buf, acc, ones_v, cacc = rest
        else:
            out_sum, src_v, dst_v, buf, acc = rest
        c = lax.axis_index("c")
        s = lax.axis_index("s")
        wid = c * NS + s
        pltpu.sync_copy(src3.at[wid], src_v)
        pltpu.sync_copy(dst3.at[wid], dst_v)
        pltpu.sync_copy(zrow, acc.at[pl.ds(s * RPT, RPT)])
        if with_counts:
            pltpu.sync_copy(ones_h, ones_v)
            pltpu.sync_copy(zcnt, cacc.at[pl.ds(s * RPT, RPT)])
        plsc.subcore_barrier()

        def step(j, carry):
            pltpu.sync_copy(table.at[src_v.at[j]], buf)
            pltpu.sync_copy(buf, acc.at[dst_v.at[j]], add=True)
            if with_counts:
                pltpu.sync_copy(ones_v, cacc.at[dst_v.at[j]], add=True)
            return carry

        lax.fori_loop(0, NCHUNK, step, 0)
        plsc.subcore_barrier()
        pltpu.sync_copy(acc.at[pl.ds(s * RPT, RPT)],
                        out_sum.at[c, pl.ds(s * RPT, RPT)])
        if with_counts:
            pltpu.sync_copy(cacc.at[pl.ds(s * RPT, RPT)],
                            out_cnt.at[c, pl.ds(s * RPT, RPT)])

    return pl.kernel(body, out_type=out_type, mesh=mesh,
                     scratch_types=scratch)


_agg_with_counts = _make_agg(True)
_agg_plain = _make_agg(False)


_BLK = 1024


def _linear_body(x_ref, wt_ref, b_ref, o_ref):
    o_ref[...] = (jnp.dot(x_ref[...], wt_ref[...],
                          preferred_element_type=jnp.float32) + b_ref[...])


def _linear(x, wt, b):
    return pl.pallas_call(
        _linear_body,
        grid=(NPAD // _BLK,),
        in_specs=[pl.BlockSpec((_BLK, F), lambda i: (i, 0)),
                  pl.BlockSpec((F, F), lambda i: (0, 0)),
                  pl.BlockSpec((1, F), lambda i: (0, 0))],
        out_specs=pl.BlockSpec((_BLK, F), lambda i: (i, 0)),
        out_shape=jax.ShapeDtypeStruct((NPAD, F), jnp.float32),
    )(x, wt, b)


def _combine_mid_body(p_ref, cnt_ref, h_ref, wlt_ref, bl_ref, wrt_ref, o_ref):
    pm = p_ref[...]
    cm = cnt_ref[...]
    cnt = cm[0][:, :1] + cm[1][:, :1]
    mean = (pm[0] + pm[1]) / jnp.clip(cnt, 1.0, None)
    h = h_ref[...]
    z = (jnp.dot(mean, wlt_ref[...], preferred_element_type=jnp.float32)
         + bl_ref[...]
         + jnp.dot(h, wrt_ref[...], preferred_element_type=jnp.float32))
    o_ref[...] = jnp.maximum(z, 0.0)


def _combine_mid(p, cnt, h, wlt, bl, wrt):
    return pl.pallas_call(
        _combine_mid_body,
        grid=(NPAD // _BLK,),
        in_specs=[pl.BlockSpec((NC, _BLK, F), lambda i: (0, i, 0)),
                  pl.BlockSpec((NC, _BLK, 16), lambda i: (0, i, 0)),
                  pl.BlockSpec((_BLK, F), lambda i: (i, 0)),
                  pl.BlockSpec((F, F), lambda i: (0, 0)),
                  pl.BlockSpec((1, F), lambda i: (0, 0)),
                  pl.BlockSpec((F, F), lambda i: (0, 0))],
        out_specs=pl.BlockSpec((_BLK, F), lambda i: (i, 0)),
        out_shape=jax.ShapeDtypeStruct((NPAD, F), jnp.float32),
    )(p, cnt, h, wlt, bl, wrt)


def _combine_last_body(p_ref, cnt_ref, h_ref, wlt_ref, bl_ref, wrt_ref,
                       wpt_ref, bp_ref, o_ref):
    pm = p_ref[...]
    cm = cnt_ref[...]
    cnt = cm[0][:, :1] + cm[1][:, :1]
    mean = (pm[0] + pm[1]) / jnp.clip(cnt, 1.0, None)
    h = h_ref[...]
    z = (jnp.dot(mean, wlt_ref[...], preferred_element_type=jnp.float32)
         + bl_ref[...]
         + jnp.dot(h, wrt_ref[...], preferred_element_type=jnp.float32))
    h2 = jnp.maximum(z, 0.0)
    logits = (jnp.dot(h2, wpt_ref[...], preferred_element_type=jnp.float32)
              + bp_ref[...])
    m = jnp.max(logits, axis=-1, keepdims=True)
    sh = logits - m
    o_ref[...] = sh - jnp.log(jnp.sum(jnp.exp(sh), axis=-1, keepdims=True))


def _combine_last(p, cnt, h, wlt, bl, wrt, wpt, bp):
    return pl.pallas_call(
        _combine_last_body,
        grid=(NPAD // _BLK,),
        in_specs=[pl.BlockSpec((NC, _BLK, F), lambda i: (0, i, 0)),
                  pl.BlockSpec((NC, _BLK, 16), lambda i: (0, i, 0)),
                  pl.BlockSpec((_BLK, F), lambda i: (i, 0)),
                  pl.BlockSpec((F, F), lambda i: (0, 0)),
                  pl.BlockSpec((1, F), lambda i: (0, 0)),
                  pl.BlockSpec((F, F), lambda i: (0, 0)),
                  pl.BlockSpec((F, F), lambda i: (0, 0)),
                  pl.BlockSpec((1, F), lambda i: (0, 0))],
        out_specs=pl.BlockSpec((_BLK, F), lambda i: (i, 0)),
        out_shape=jax.ShapeDtypeStruct((NPAD, F), jnp.float32),
    )(p, cnt, h, wlt, bl, wrt, wpt, bp)


def kernel(x, edge_index, W_pre, b_pre, Wl1, bl1, Wr1, Wl2, bl2, Wr2,
           W_post, b_post):
    src3 = edge_index[0].reshape(NW, NCHUNK, CHUNK)
    dst3 = edge_index[1].reshape(NW, NCHUNK, CHUNK)
    zrow = jnp.zeros((RPT, F), jnp.float32)
    zcnt = jnp.zeros((RPT, 16), jnp.float32)
    ones_h = jnp.ones((CHUNK, 16), jnp.float32)

    x_pad = jnp.pad(x, ((0, NPAD - N), (0, 0)))
    wpt = jnp.pad(W_post.T, ((0, 0), (0, F - C)))
    bp = jnp.pad(b_post, (0, F - C), constant_values=-1e30)

    h0 = _linear(x_pad, W_pre.T, b_pre[None])
    s1, cnt = _agg_with_counts(h0, src3, dst3, zrow, zcnt, ones_h)
    h1 = _combine_mid(s1, cnt, h0, Wl1.T, bl1[None], Wr1.T)
    s2, _ = _agg_plain(h1, src3, dst3, zrow, zcnt, ones_h), None
    s2 = s2[0] if isinstance(s2, (list, tuple)) else s2
    out = _combine_last(s2, cnt, h1, Wl2.T, bl2[None], Wr2.T, wpt, bp[None])
    return out[:N, :C]


# speedup vs baseline: 6.9403x; 6.9403x over previous
import jax
import jax.numpy as jnp
from jax import lax
from jax.experimental import pallas as pl
from jax.experimental.pallas import tpu as pltpu
from jax.experimental.pallas import tpu_sc as plsc

N = 10000
E = 320000
F = 128
C = 47

NC = 2
NS = 16
NW = NC * NS
EW = E // NW
CHUNK = 80
NCHUNK = EW // CHUNK
SUB = 25
NSUB = NCHUNK // SUB
CPS = EW // NSUB
NPAD = 10240
RPT = NPAD // NS

_mesh = plsc.VectorSubcoreMesh(core_axis_name="c", subcore_axis_name="s",
                               num_cores=NC, num_subcores=NS)


def _make_agg():
    scratch = [
        pltpu.VMEM((SUB, CHUNK), jnp.int32),
        pltpu.VMEM((SUB, CHUNK), jnp.int32),
        pltpu.VMEM((CHUNK, F), jnp.float32),
        pltpu.VMEM_SHARED((NPAD, F), jnp.float32),
    ]

    def body(table, src3, dst3, zrow, out_sum, src_v, dst_v, buf, acc):
        c = lax.axis_index("c")
        s = lax.axis_index("s")
        wid = c * NS + s
        pltpu.sync_copy(zrow, buf)
        for k in range(RPT // CHUNK):
            pltpu.sync_copy(buf, acc.at[pl.ds(s * RPT + k * CHUNK, CHUNK)])
        plsc.subcore_barrier()

        def step(j, carry):
            pltpu.sync_copy(table.at[src_v.at[j]], buf)
            pltpu.sync_copy(buf, acc.at[dst_v.at[j]], add=True)
            return carry

        def super_step(o, carry):
            pltpu.sync_copy(src3.at[wid, o], src_v)
            pltpu.sync_copy(dst3.at[wid, o], dst_v)
            lax.fori_loop(0, SUB, step, carry)
            return carry

        lax.fori_loop(0, NSUB, super_step, 0)
        plsc.subcore_barrier()
        for k in range(RPT // CHUNK):
            pltpu.sync_copy(acc.at[pl.ds(s * RPT + k * CHUNK, CHUNK)], buf)
            pltpu.sync_copy(
                buf, out_sum.at[c, pl.ds(s * RPT + k * CHUNK, CHUNK)])

    return pl.kernel(
        body,
        out_type=jax.ShapeDtypeStruct((NC, NPAD, F), jnp.float32),
        mesh=_mesh, scratch_types=scratch)


_agg = _make_agg()


def _make_count():
    scratch = [
        pltpu.VMEM((CPS,), jnp.int32),
        pltpu.VMEM((NPAD,), jnp.float32),
        pltpu.VMEM((NPAD,), jnp.float32),
        pltpu.VMEM_SHARED((NS * NPAD,), jnp.float32),
    ]

    def body(dstf, zpriv, out, dst_v, cnt_priv, gbuf, cshared):
        c = lax.axis_index("c")
        s = lax.axis_index("s")
        wid = c * NS + s
        pltpu.sync_copy(zpriv, cnt_priv)
        one16 = jnp.ones((16,), jnp.float32)

        def step(i, carry):
            idx = dst_v[pl.ds(i * 16, 16)]
            plsc.addupdate_scatter(cnt_priv, [idx], one16)
            return carry

        def super_step(o, carry):
            pltpu.sync_copy(dstf.at[pl.ds(wid * EW + o * CPS, CPS)], dst_v)
            lax.fori_loop(0, CPS // 16, step, carry)
            return carry

        lax.fori_loop(0, NSUB, super_step, 0)
        for r in range(NS):
            pltpu.sync_copy(cnt_priv.at[pl.ds(r * RPT, RPT)],
                            cshared.at[pl.ds((s * NS + r) * RPT, RPT)])
        plsc.subcore_barrier()
        for t in range(NS):
            pltpu.sync_copy(cshared.at[pl.ds((t * NS + s) * RPT, RPT)],
                            gbuf.at[pl.ds(t * RPT, RPT)])

        def red(i, carry):
            v = gbuf[pl.ds(i * 16, 16)]
            for t in range(1, NS):
                v = v + gbuf[pl.ds(t * RPT + i * 16, 16)]
            cnt_priv[pl.ds(i * 16, 16)] = v
            return carry

        lax.fori_loop(0, RPT // 16, red, 0)
        pltpu.sync_copy(cnt_priv.at[pl.ds(0, RPT)],
                        out.at[pl.ds(c * NPAD + s * RPT, RPT)])

    return pl.kernel(
        body,
        out_type=jax.ShapeDtypeStruct((NC * NPAD,), jnp.float32),
        mesh=_mesh,
        compiler_params=pltpu.CompilerParams(needs_layout_passes=False),
        scratch_types=scratch)


_count = _make_count()


_BLK = 1024


def _linear_body(x_ref, wt_ref, b_ref, o_ref):
    o_ref[...] = (jnp.dot(x_ref[...], wt_ref[...],
                          preferred_element_type=jnp.float32) + b_ref[...])


def _linear(x, wt, b):
    return pl.pallas_call(
        _linear_body,
        grid=(NPAD // _BLK,),
        in_specs=[pl.BlockSpec((_BLK, F), lambda i: (i, 0)),
                  pl.BlockSpec((F, F), lambda i: (0, 0)),
                  pl.BlockSpec((1, F), lambda i: (0, 0))],
        out_specs=pl.BlockSpec((_BLK, F), lambda i: (i, 0)),
        out_shape=jax.ShapeDtypeStruct((NPAD, F), jnp.float32),
    )(x, wt, b)


def _mean_mm(p_ref, cnt_ref, h_ref, wlt_ref, bl_ref, wrt_ref):
    pm = p_ref[...]
    cm = cnt_ref[...]
    cnt = (cm[0] + cm[1])[:, None]
    mean = (pm[0] + pm[1]) / jnp.clip(cnt, 1.0, None)
    return (jnp.dot(mean, wlt_ref[...], preferred_element_type=jnp.float32)
            + bl_ref[...]
            + jnp.dot(h_ref[...], wrt_ref[...],
                      preferred_element_type=jnp.float32))


def _combine_mid_body(p_ref, cnt_ref, h_ref, wlt_ref, bl_ref, wrt_ref, o_ref):
    o_ref[...] = jnp.maximum(
        _mean_mm(p_ref, cnt_ref, h_ref, wlt_ref, bl_ref, wrt_ref), 0.0)


def _combine_last_body(p_ref, cnt_ref, h_ref, wlt_ref, bl_ref, wrt_ref,
                       wpt_ref, bp_ref, o_ref):
    h2 = jnp.maximum(
        _mean_mm(p_ref, cnt_ref, h_ref, wlt_ref, bl_ref, wrt_ref), 0.0)
    logits = (jnp.dot(h2, wpt_ref[...], preferred_element_type=jnp.float32)
              + bp_ref[...])
    m = jnp.max(logits, axis=-1, keepdims=True)
    sh = logits - m
    o_ref[...] = sh - jnp.log(jnp.sum(jnp.exp(sh), axis=-1, keepdims=True))


_SPECS_MID = [pl.BlockSpec((NC, _BLK, F), lambda i: (0, i, 0)),
              pl.BlockSpec((NC, _BLK), lambda i: (0, i)),
              pl.BlockSpec((_BLK, F), lambda i: (i, 0)),
              pl.BlockSpec((F, F), lambda i: (0, 0)),
              pl.BlockSpec((1, F), lambda i: (0, 0)),
              pl.BlockSpec((F, F), lambda i: (0, 0))]


def _combine_mid(p, cnt, h, wlt, bl, wrt):
    return pl.pallas_call(
        _combine_mid_body,
        grid=(NPAD // _BLK,),
        in_specs=_SPECS_MID,
        out_specs=pl.BlockSpec((_BLK, F), lambda i: (i, 0)),
        out_shape=jax.ShapeDtypeStruct((NPAD, F), jnp.float32),
    )(p, cnt, h, wlt, bl, wrt)


def _combine_last(p, cnt, h, wlt, bl, wrt, wpt, bp):
    return pl.pallas_call(
        _combine_last_body,
        grid=(NPAD // _BLK,),
        in_specs=_SPECS_MID + [pl.BlockSpec((F, F), lambda i: (0, 0)),
                               pl.BlockSpec((1, F), lambda i: (0, 0))],
        out_specs=pl.BlockSpec((_BLK, F), lambda i: (i, 0)),
        out_shape=jax.ShapeDtypeStruct((NPAD, F), jnp.float32),
    )(p, cnt, h, wlt, bl, wrt, wpt, bp)


def kernel(x, edge_index, W_pre, b_pre, Wl1, bl1, Wr1, Wl2, bl2, Wr2,
           W_post, b_post):
    src3 = edge_index[0].reshape(NW, NSUB, SUB, CHUNK)
    dst3 = edge_index[1].reshape(NW, NSUB, SUB, CHUNK)
    dstf = edge_index[1]
    zrow = jnp.zeros((CHUNK, F), jnp.float32)
    zpriv = jnp.zeros((NPAD,), jnp.float32)

    x_pad = jnp.pad(x, ((0, NPAD - N), (0, 0)))
    wpt = jnp.pad(W_post.T, ((0, 0), (0, F - C)))
    bp = jnp.pad(b_post, (0, F - C), constant_values=-1e30)

    cnt = _count(dstf, zpriv).reshape(NC, NPAD)
    h0 = _linear(x_pad, W_pre.T, b_pre[None])
    s1 = _agg(h0, src3, dst3, zrow)
    h1 = _combine_mid(s1, cnt, h0, Wl1.T, bl1[None], Wr1.T)
    s2 = _agg(h1, src3, dst3, zrow)
    out = _combine_last(s2, cnt, h1, Wl2.T, bl2[None], Wr2.T, wpt, bp[None])
    return out[:N, :C]


# speedup vs baseline: 8.5900x; 1.2377x over previous
import jax
import jax.numpy as jnp
from jax import lax
from jax.experimental import pallas as pl
from jax.experimental.pallas import tpu as pltpu
from jax.experimental.pallas import tpu_sc as plsc

N = 10000
E = 320000
F = 128
C = 47

NC = 2
NS = 16
NW = NC * NS
EW = E // NW
CHUNK = 80
NCHUNK = EW // CHUNK
SUB = 25
NSUB = NCHUNK // SUB
CPS = EW // NSUB
NPAD = 10240
RPT = NPAD // NS

_mesh = plsc.VectorSubcoreMesh(core_axis_name="c", subcore_axis_name="s",
                               num_cores=NC, num_subcores=NS)


def _make_agg():
    scratch = [
        pltpu.VMEM((SUB, CHUNK), jnp.int32),
        pltpu.VMEM((SUB, CHUNK), jnp.int32),
        pltpu.VMEM((CHUNK, F), jnp.float32),
        pltpu.VMEM((CHUNK, F), jnp.float32),
        pltpu.VMEM_SHARED((NPAD, F), jnp.float32),
        pltpu.SemaphoreType.DMA,
        pltpu.SemaphoreType.DMA,
    ]

    def body(table, src3, dst3, zrow, out_sum, src_v, dst_v, buf0, buf1,
             acc, sem0, sem1):
        c = lax.axis_index("c")
        s = lax.axis_index("s")
        wid = c * NS + s
        pltpu.sync_copy(zrow, buf0)
        for k in range(RPT // CHUNK):
            pltpu.sync_copy(buf0, acc.at[pl.ds(s * RPT + k * CHUNK, CHUNK)])
        plsc.subcore_barrier()

        def step(j, carry):
            @pl.when(j % 2 == 0)
            def _():
                pltpu.make_async_copy(
                    table.at[src_v.at[j]], buf0, sem0).wait()

                @pl.when(j < SUB - 1)
                def _():
                    pltpu.async_copy(
                        table.at[src_v.at[j + 1]], buf1, sem1)

                pltpu.sync_copy(buf0, acc.at[dst_v.at[j]], add=True)

            @pl.when(j % 2 == 1)
            def _():
                pltpu.make_async_copy(
                    table.at[src_v.at[j]], buf1, sem1).wait()

                @pl.when(j < SUB - 1)
                def _():
                    pltpu.async_copy(
                        table.at[src_v.at[j + 1]], buf0, sem0)

                pltpu.sync_copy(buf1, acc.at[dst_v.at[j]], add=True)

            return carry

        def super_step(o, carry):
            pltpu.sync_copy(src3.at[wid, o], src_v)
            pltpu.sync_copy(dst3.at[wid, o], dst_v)
            pltpu.async_copy(table.at[src_v.at[0]], buf0, sem0)
            return lax.fori_loop(0, SUB, step, carry)

        lax.fori_loop(0, NSUB, super_step, 0)
        plsc.subcore_barrier()
        for k in range(RPT // CHUNK):
            pltpu.sync_copy(acc.at[pl.ds(s * RPT + k * CHUNK, CHUNK)], buf0)
            pltpu.sync_copy(
                buf0, out_sum.at[c, pl.ds(s * RPT + k * CHUNK, CHUNK)])

    return pl.kernel(
        body,
        out_type=jax.ShapeDtypeStruct((NC, NPAD, F), jnp.float32),
        mesh=_mesh, scratch_types=scratch)


_agg = _make_agg()


def _make_count():
    scratch = [
        pltpu.VMEM((CPS,), jnp.int32),
        pltpu.VMEM((NPAD,), jnp.float32),
        pltpu.VMEM((NPAD,), jnp.float32),
        pltpu.VMEM_SHARED((NS * NPAD,), jnp.float32),
    ]

    def body(dstf, zpriv, out, dst_v, cnt_priv, gbuf, cshared):
        c = lax.axis_index("c")
        s = lax.axis_index("s")
        wid = c * NS + s
        pltpu.sync_copy(zpriv, cnt_priv)
        one16 = jnp.ones((16,), jnp.float32)

        def step(i, carry):
            idx = dst_v[pl.ds(i * 16, 16)]
            plsc.addupdate_scatter(cnt_priv, [idx], one16)
            return carry

        def super_step(o, carry):
            pltpu.sync_copy(dstf.at[pl.ds(wid * EW + o * CPS, CPS)], dst_v)
            lax.fori_loop(0, CPS // 16, step, carry)
            return carry

        lax.fori_loop(0, NSUB, super_step, 0)
        for r in range(NS):
            pltpu.sync_copy(cnt_priv.at[pl.ds(r * RPT, RPT)],
                            cshared.at[pl.ds((s * NS + r) * RPT, RPT)])
        plsc.subcore_barrier()
        for t in range(NS):
            pltpu.sync_copy(cshared.at[pl.ds((t * NS + s) * RPT, RPT)],
                            gbuf.at[pl.ds(t * RPT, RPT)])

        def red(i, carry):
            v = gbuf[pl.ds(i * 16, 16)]
            for t in range(1, NS):
                v = v + gbuf[pl.ds(t * RPT + i * 16, 16)]
            cnt_priv[pl.ds(i * 16, 16)] = v
            return carry

        lax.fori_loop(0, RPT // 16, red, 0)
        pltpu.sync_copy(cnt_priv.at[pl.ds(0, RPT)],
                        out.at[pl.ds(c * NPAD + s * RPT, RPT)])

    return pl.kernel(
        body,
        out_type=jax.ShapeDtypeStruct((NC * NPAD,), jnp.float32),
        mesh=_mesh,
        compiler_params=pltpu.CompilerParams(needs_layout_passes=False),
        scratch_types=scratch)


_count = _make_count()


_BLK = 1024


def _linear_body(x_ref, wt_ref, b_ref, o_ref):
    o_ref[...] = (jnp.dot(x_ref[...], wt_ref[...],
                          preferred_element_type=jnp.float32) + b_ref[...])


def _linear(x, wt, b):
    return pl.pallas_call(
        _linear_body,
        grid=(NPAD // _BLK,),
        in_specs=[pl.BlockSpec((_BLK, F), lambda i: (i, 0)),
                  pl.BlockSpec((F, F), lambda i: (0, 0)),
                  pl.BlockSpec((1, F), lambda i: (0, 0))],
        out_specs=pl.BlockSpec((_BLK, F), lambda i: (i, 0)),
        out_shape=jax.ShapeDtypeStruct((NPAD, F), jnp.float32),
    )(x, wt, b)


def _mean_mm(p_ref, cnt_ref, h_ref, wlt_ref, bl_ref, wrt_ref):
    pm = p_ref[...]
    cm = cnt_ref[...]
    cnt = (cm[0] + cm[1])[:, None]
    mean = (pm[0] + pm[1]) / jnp.clip(cnt, 1.0, None)
    return (jnp.dot(mean, wlt_ref[...], preferred_element_type=jnp.float32)
            + bl_ref[...]
            + jnp.dot(h_ref[...], wrt_ref[...],
                      preferred_element_type=jnp.float32))


def _combine_mid_body(p_ref, cnt_ref, h_ref, wlt_ref, bl_ref, wrt_ref, o_ref):
    o_ref[...] = jnp.maximum(
        _mean_mm(p_ref, cnt_ref, h_ref, wlt_ref, bl_ref, wrt_ref), 0.0)


def _combine_last_body(p_ref, cnt_ref, h_ref, wlt_ref, bl_ref, wrt_ref,
                       wpt_ref, bp_ref, o_ref):
    h2 = jnp.maximum(
        _mean_mm(p_ref, cnt_ref, h_ref, wlt_ref, bl_ref, wrt_ref), 0.0)
    logits = (jnp.dot(h2, wpt_ref[...], preferred_element_type=jnp.float32)
              + bp_ref[...])
    m = jnp.max(logits, axis=-1, keepdims=True)
    sh = logits - m
    o_ref[...] = sh - jnp.log(jnp.sum(jnp.exp(sh), axis=-1, keepdims=True))


_SPECS_MID = [pl.BlockSpec((NC, _BLK, F), lambda i: (0, i, 0)),
              pl.BlockSpec((NC, _BLK), lambda i: (0, i)),
              pl.BlockSpec((_BLK, F), lambda i: (i, 0)),
              pl.BlockSpec((F, F), lambda i: (0, 0)),
              pl.BlockSpec((1, F), lambda i: (0, 0)),
              pl.BlockSpec((F, F), lambda i: (0, 0))]


def _combine_mid(p, cnt, h, wlt, bl, wrt):
    return pl.pallas_call(
        _combine_mid_body,
        grid=(NPAD // _BLK,),
        in_specs=_SPECS_MID,
        out_specs=pl.BlockSpec((_BLK, F), lambda i: (i, 0)),
        out_shape=jax.ShapeDtypeStruct((NPAD, F), jnp.float32),
    )(p, cnt, h, wlt, bl, wrt)


def _combine_last(p, cnt, h, wlt, bl, wrt, wpt, bp):
    return pl.pallas_call(
        _combine_last_body,
        grid=(NPAD // _BLK,),
        in_specs=_SPECS_MID + [pl.BlockSpec((F, F), lambda i: (0, 0)),
                               pl.BlockSpec((1, F), lambda i: (0, 0))],
        out_specs=pl.BlockSpec((_BLK, F), lambda i: (i, 0)),
        out_shape=jax.ShapeDtypeStruct((NPAD, F), jnp.float32),
    )(p, cnt, h, wlt, bl, wrt, wpt, bp)


def kernel(x, edge_index, W_pre, b_pre, Wl1, bl1, Wr1, Wl2, bl2, Wr2,
           W_post, b_post):
    src3 = edge_index[0].reshape(NW, NSUB, SUB, CHUNK)
    dst3 = edge_index[1].reshape(NW, NSUB, SUB, CHUNK)
    dstf = edge_index[1]
    zrow = jnp.zeros((CHUNK, F), jnp.float32)
    zpriv = jnp.zeros((NPAD,), jnp.float32)

    x_pad = jnp.pad(x, ((0, NPAD - N), (0, 0)))
    wpt = jnp.pad(W_post.T, ((0, 0), (0, F - C)))
    bp = jnp.pad(b_post, (0, F - C), constant_values=-1e30)

    cnt = _count(dstf, zpriv).reshape(NC, NPAD)
    h0 = _linear(x_pad, W_pre.T, b_pre[None])
    s1 = _agg(h0, src3, dst3, zrow)
    h1 = _combine_mid(s1, cnt, h0, Wl1.T, bl1[None], Wr1.T)
    s2 = _agg(h1, src3, dst3, zrow)
    out = _combine_last(s2, cnt, h1, Wl2.T, bl2[None], Wr2.T, wpt, bp[None])
    return out[:N, :C]
